# Initial kernel scaffold; baseline (speedup 1.0000x reference)
import jax
import jax.numpy as jnp
from jax.experimental import pallas as pl


def kernel(atomic_positions, atomic_numbers, emb, fw1_W, fw1_b, fw2_W, fw2_b, in2f_W, in2f_b, f2out_W, f2out_b, dense_W, dense_b, e1_W, e1_b, e2_W, e2_b):
    raise NotImplementedError("write your pallas kernel here")


# speedup vs baseline: 2.8727x; 2.8727x over previous
import functools

import jax
import jax.numpy as jnp
from jax.experimental import pallas as pl
from jax.experimental.pallas import tpu as pltpu

N_BLOCKS = 3
N_ATOM_BASIS = 128
N_FILTERS = 64
N_GAUSS = 25
MAX_Z = 5
CUTOFF = 5.0
N_CONFIGS = 16
N_ATOMS = 128

_DELTA = CUTOFF / (N_GAUSS - 1)
_ALPHA = 0.5 / (_DELTA * _DELTA)

_TI = 32
_NT = N_ATOMS // _TI

_F32 = jnp.float32


def _dot(a, b, dn=(((1,), (0,)), ((), ()))):
    return jax.lax.dot_general(a, b, dn, preferred_element_type=_F32)


def _body(pos_ref, an_ref, emb_ref, fw1w_ref, fw1b_ref, fw2w_ref, fw2b_ref,
          i2fw_ref, i2fb_ref, f2ow_ref, f2ob_ref, dw_ref, db_ref,
          e1w_ref, e1b_ref, e2w_ref, e2b_ref, out_ref, w_scr):
    A = N_ATOMS
    pos = pos_ref[0]
    dot = _dot(pos, pos, (((1,), (1,)), ((), ())))
    psq = pos * pos
    n2c = jnp.sum(psq, axis=1, keepdims=True)
    ones13 = jnp.ones((1, 3), _F32)
    n2r = _dot(ones13, psq, (((1,), (1,)), ((), ())))
    r2 = n2c + n2r - 2.0 * dot
    ii = jax.lax.broadcasted_iota(jnp.int32, (A, A), 0)
    jj = jax.lax.broadcasted_iota(jnp.int32, (A, A), 1)
    r2 = jnp.where(ii == jj, 0.0, jnp.maximum(r2, 0.0))
    r = jnp.sqrt(r2)

    centers3 = jax.lax.broadcasted_iota(
        jnp.int32, (1, 1, N_GAUSS), 2).astype(_F32) * _DELTA
    centers2 = jax.lax.broadcasted_iota(
        jnp.int32, (1, N_GAUSS), 1).astype(_F32) * _DELTA

    for t in range(_NT):
        rt = r[t * _TI:(t + 1) * _TI, :]
        f3 = jnp.exp(-_ALPHA * (rt[:, :, None] - centers3) ** 2)
        f2 = f3.reshape(_TI * A, N_GAUSS)
        for b in range(N_BLOCKS):
            h = jnp.tanh(_dot(f2, fw1w_ref[b]) + fw1b_ref[b])
            w = _dot(h, fw2w_ref[b]) + fw2b_ref[b]
            w_scr[b, t * _TI * A:(t + 1) * _TI * A, :] = w

    f0 = jnp.exp(-_ALPHA * centers2 * centers2)
    w_self = []
    for b in range(N_BLOCKS):
        h0 = jnp.tanh(_dot(f0, fw1w_ref[b]) + fw1b_ref[b])
        w_self.append(_dot(h0, fw2w_ref[b]) + fw2b_ref[b])

    an = an_ref[0]
    onehot = (an == jax.lax.broadcasted_iota(jnp.int32, (A, MAX_Z), 1)
              ).astype(_F32)
    x = _dot(onehot, emb_ref[...])

    for b in range(N_BLOCKS):
        y = _dot(x, i2fw_ref[b]) + i2fb_ref[b]
        accs = []
        for t in range(_NT):
            wt = w_scr[b, t * _TI * A:(t + 1) * _TI * A, :]
            w3 = wt.reshape(_TI, A, N_FILTERS)
            accs.append(jnp.sum(w3 * y[None, :, :], axis=1))
        agg = jnp.concatenate(accs, axis=0) - w_self[b] * y
        y2 = jnp.tanh(_dot(agg, f2ow_ref[b]) + f2ob_ref[b])
        x = x + _dot(y2, dw_ref[b]) + db_ref[b]

    t1 = jnp.tanh(_dot(x, e1w_ref[...]) + e1b_ref[...])
    o = _dot(t1, e2w_ref[...]) + e2b_ref[...]
    out_ref[0] = jnp.sum(o, axis=0, keepdims=True)


@jax.jit
def kernel(atomic_positions, atomic_numbers, emb, fw1_W, fw1_b, fw2_W, fw2_b,
           in2f_W, in2f_b, f2out_W, f2out_b, dense_W, dense_b,
           e1_W, e1_b, e2_W, e2_b):
    C, A = atomic_positions.shape[0], atomic_positions.shape[1]
    an3 = atomic_numbers.astype(jnp.int32).reshape(C, A, 1)

    def rep(shape):
        nd = len(shape)
        return pl.BlockSpec(shape, lambda c, _n=nd: (0,) * _n)

    in_specs = [
            pl.BlockSpec((1, A, 3), lambda c: (c, 0, 0)),
            pl.BlockSpec((1, A, 1), lambda c: (c, 0, 0)),
            rep(emb.shape),
            rep(fw1_W.shape), rep(fw1_b.shape),
            rep(fw2_W.shape), rep(fw2_b.shape),
            rep(in2f_W.shape), rep(in2f_b.shape),
            rep(f2out_W.shape), rep(f2out_b.shape),
            rep(dense_W.shape), rep(dense_b.shape),
            rep(e1_W.shape), rep(e1_b.shape),
            rep(e2_W.shape), rep(e2_b.shape),
    ]
    out = pl.pallas_call(
        _body,
        grid=(C,),
        in_specs=in_specs,
        out_specs=pl.BlockSpec((1, 1, 1), lambda c: (c, 0, 0)),
        out_shape=jax.ShapeDtypeStruct((C, 1, 1), _F32),
        scratch_shapes=[pltpu.VMEM((N_BLOCKS, A * A, N_FILTERS), _F32)],
        compiler_params=pltpu.CompilerParams(
            dimension_semantics=("arbitrary",)),
    )(atomic_positions, an3, emb, fw1_W, fw1_b, fw2_W, fw2_b,
      in2f_W, in2f_b, f2out_W, f2out_b, dense_W, dense_b,
      e1_W, e1_b, e2_W, e2_b)
    return out.reshape(C, 1)


# speedup vs baseline: 2.8780x; 1.0019x over previous
import functools

import jax
import jax.numpy as jnp
from jax.experimental import pallas as pl
from jax.experimental.pallas import tpu as pltpu

N_BLOCKS = 3
N_ATOM_BASIS = 128
N_FILTERS = 64
N_GAUSS = 25
MAX_Z = 5
CUTOFF = 5.0
N_CONFIGS = 16
N_ATOMS = 128

_DELTA = CUTOFF / (N_GAUSS - 1)
_ALPHA = 0.5 / (_DELTA * _DELTA)

_TI = 32
_NT = N_ATOMS // _TI

_F32 = jnp.float32


def _dot(a, b, dn=(((1,), (0,)), ((), ()))):
    return jax.lax.dot_general(a, b, dn, preferred_element_type=_F32)


def _body(pos_ref, an_ref, emb_ref, fw1w_ref, fw1b_ref, fw2w_ref, fw2b_ref,
          i2fw_ref, i2fb_ref, f2ow_ref, f2ob_ref, dw_ref, db_ref,
          e1w_ref, e1b_ref, e2w_ref, e2b_ref, out_ref, w_scr):
    A = N_ATOMS
    pos = pos_ref[0]
    dot = _dot(pos, pos, (((1,), (1,)), ((), ())))
    psq = pos * pos
    n2c = jnp.sum(psq, axis=1, keepdims=True)
    ones13 = jnp.ones((1, 3), _F32)
    n2r = _dot(ones13, psq, (((1,), (1,)), ((), ())))
    r2 = n2c + n2r - 2.0 * dot
    ii = jax.lax.broadcasted_iota(jnp.int32, (A, A), 0)
    jj = jax.lax.broadcasted_iota(jnp.int32, (A, A), 1)
    r2 = jnp.where(ii == jj, 0.0, jnp.maximum(r2, 0.0))
    r = jnp.sqrt(r2)

    centers3 = jax.lax.broadcasted_iota(
        jnp.int32, (1, 1, N_GAUSS), 2).astype(_F32) * _DELTA
    centers2 = jax.lax.broadcasted_iota(
        jnp.int32, (1, N_GAUSS), 1).astype(_F32) * _DELTA

    for t in range(_NT):
        rt = r[t * _TI:(t + 1) * _TI, :]
        f3 = jnp.exp(-_ALPHA * (rt[:, :, None] - centers3) ** 2)
        f2 = f3.reshape(_TI * A, N_GAUSS).astype(jnp.bfloat16)
        for b in range(N_BLOCKS):
            h = jnp.tanh(_dot(f2, fw1w_ref[b].astype(jnp.bfloat16))
                         + fw1b_ref[b]).astype(jnp.bfloat16)
            w = _dot(h, fw2w_ref[b].astype(jnp.bfloat16)) + fw2b_ref[b]
            w_scr[b, t * _TI * A:(t + 1) * _TI * A, :] = w

    f0 = jnp.exp(-_ALPHA * centers2 * centers2)
    w_self = []
    for b in range(N_BLOCKS):
        h0 = jnp.tanh(_dot(f0, fw1w_ref[b]) + fw1b_ref[b])
        w_self.append(_dot(h0, fw2w_ref[b]) + fw2b_ref[b])

    an = an_ref[0]
    onehot = (an == jax.lax.broadcasted_iota(jnp.int32, (A, MAX_Z), 1)
              ).astype(_F32)
    x = _dot(onehot, emb_ref[...])

    for b in range(N_BLOCKS):
        y = _dot(x, i2fw_ref[b]) + i2fb_ref[b]
        accs = []
        for t in range(_NT):
            wt = w_scr[b, t * _TI * A:(t + 1) * _TI * A, :]
            w3 = wt.reshape(_TI, A, N_FILTERS)
            accs.append(jnp.sum(w3 * y[None, :, :], axis=1))
        agg = jnp.concatenate(accs, axis=0) - w_self[b] * y
        y2 = jnp.tanh(_dot(agg, f2ow_ref[b]) + f2ob_ref[b])
        x = x + _dot(y2, dw_ref[b]) + db_ref[b]

    t1 = jnp.tanh(_dot(x, e1w_ref[...]) + e1b_ref[...])
    o = _dot(t1, e2w_ref[...]) + e2b_ref[...]
    out_ref[0] = jnp.sum(o, axis=0, keepdims=True)


@jax.jit
def kernel(atomic_positions, atomic_numbers, emb, fw1_W, fw1_b, fw2_W, fw2_b,
           in2f_W, in2f_b, f2out_W, f2out_b, dense_W, dense_b,
           e1_W, e1_b, e2_W, e2_b):
    C, A = atomic_positions.shape[0], atomic_positions.shape[1]
    an3 = atomic_numbers.astype(jnp.int32).reshape(C, A, 1)

    def rep(shape):
        nd = len(shape)
        return pl.BlockSpec(shape, lambda c, _n=nd: (0,) * _n)

    in_specs = [
            pl.BlockSpec((1, A, 3), lambda c: (c, 0, 0)),
            pl.BlockSpec((1, A, 1), lambda c: (c, 0, 0)),
            rep(emb.shape),
            rep(fw1_W.shape), rep(fw1_b.shape),
            rep(fw2_W.shape), rep(fw2_b.shape),
            rep(in2f_W.shape), rep(in2f_b.shape),
            rep(f2out_W.shape), rep(f2out_b.shape),
            rep(dense_W.shape), rep(dense_b.shape),
            rep(e1_W.shape), rep(e1_b.shape),
            rep(e2_W.shape), rep(e2_b.shape),
    ]
    out = pl.pallas_call(
        _body,
        grid=(C,),
        in_specs=in_specs,
        out_specs=pl.BlockSpec((1, 1, 1), lambda c: (c, 0, 0)),
        out_shape=jax.ShapeDtypeStruct((C, 1, 1), _F32),
        scratch_shapes=[pltpu.VMEM((N_BLOCKS, A * A, N_FILTERS), _F32)],
        compiler_params=pltpu.CompilerParams(
            dimension_semantics=("arbitrary",)),
    )(atomic_positions, an3, emb, fw1_W, fw1_b, fw2_W, fw2_b,
      in2f_W, in2f_b, f2out_W, f2out_b, dense_W, dense_b,
      e1_W, e1_b, e2_W, e2_b)
    return out.reshape(C, 1)
